# Initial kernel scaffold; baseline (speedup 1.0000x reference)
import jax
import jax.numpy as jnp
from jax.experimental import pallas as pl


def kernel(x, edge_index, edge_attr, params):
    raise NotImplementedError("write your pallas kernel here")


# speedup vs baseline: 1.1992x; 1.1992x over previous
import functools

import jax
import jax.numpy as jnp
from jax import lax
from jax.experimental import pallas as pl
from jax.experimental.pallas import tpu as pltpu
from jax.experimental.pallas import tpu_sc as plsc

N = 10000
E = 320000
D = 128
DE = 16
H = 128
C = 8

NC = 2
NS = 16
NW = NC * NS
CB = 128
NCHUNK = E // CB
ROWS_PT = 640
N_TAIL = N - (NS - 1) * ROWS_PT

_f32 = jnp.float32


def _mm_relu_k(x_ref, w_ref, b_ref, o_ref):
    o_ref[...] = jnp.maximum(
        jnp.dot(x_ref[...], w_ref[...], preferred_element_type=_f32)
        + b_ref[...], 0.0)


def _in_proj(x, w, b):
    nb = 10
    return pl.pallas_call(
        _mm_relu_k,
        grid=(nb,),
        in_specs=[pl.BlockSpec((N // nb, D), lambda i: (i, 0)),
                  pl.BlockSpec((D, H), lambda i: (0, 0)),
                  pl.BlockSpec((1, H), lambda i: (0, 0))],
        out_specs=pl.BlockSpec((N // nb, H), lambda i: (i, 0)),
        out_shape=jax.ShapeDtypeStruct((N, H), _f32),
    )(x, w, b)


def _edge_proj_k(ea_ref, w_ref, b_ref, o0, o1, o2, o3):
    y = jnp.dot(ea_ref[...], w_ref[...], preferred_element_type=_f32) + b_ref[...]
    o0[...] = y[:, :H]
    o1[...] = y[:, H:2 * H]
    o2[...] = y[:, 2 * H:3 * H]
    o3[...] = y[:, 3 * H:]


def _edge_proj(ea, w, b):
    eb = 4000
    return pl.pallas_call(
        _edge_proj_k,
        grid=(E // eb,),
        in_specs=[pl.BlockSpec((eb, DE), lambda i: (i, 0)),
                  pl.BlockSpec((DE, 4 * H), lambda i: (0, 0)),
                  pl.BlockSpec((1, 4 * H), lambda i: (0, 0))],
        out_specs=[pl.BlockSpec((eb, H), lambda i: (i, 0))] * 4,
        out_shape=[jax.ShapeDtypeStruct((E, H), _f32)] * 4,
    )(ea, w, b)


def _node_mlp_k(h_ref, a0_ref, a1_ref, w1_ref, b1_ref, w2_ref, b2_ref,
                z2_ref, st_ref):
    z = h_ref[...] + (a0_ref[...] + a1_ref[...])
    t = jnp.maximum(
        jnp.dot(z, w1_ref[...], preferred_element_type=_f32) + b1_ref[...], 0.0)
    z2 = jnp.dot(t, w2_ref[...], preferred_element_type=_f32) + b2_ref[...]
    z2_ref[...] = z2
    st_ref[...] = jnp.stack([jnp.sum(z2, axis=0), jnp.sum(z2 * z2, axis=0)])[None]


def _node_mlp(h, a0, a1, w1, b1, w2, b2):
    nb = 10
    return pl.pallas_call(
        _node_mlp_k,
        grid=(nb,),
        in_specs=[pl.BlockSpec((N // nb, H), lambda i: (i, 0)),
                  pl.BlockSpec((N // nb, H), lambda i: (i, 0)),
                  pl.BlockSpec((N // nb, H), lambda i: (i, 0)),
                  pl.BlockSpec((H, 2 * H), lambda i: (0, 0)),
                  pl.BlockSpec((1, 2 * H), lambda i: (0, 0)),
                  pl.BlockSpec((2 * H, H), lambda i: (0, 0)),
                  pl.BlockSpec((1, H), lambda i: (0, 0))],
        out_specs=[pl.BlockSpec((N // nb, H), lambda i: (i, 0)),
                   pl.BlockSpec((1, 2, H), lambda i: (i, 0, 0))],
        out_shape=[jax.ShapeDtypeStruct((N, H), _f32),
                   jax.ShapeDtypeStruct((nb, 2, H), _f32)],
    )(h, a0, a1, w1, b1, w2, b2)


def _bn_k(z2_ref, st_ref, g_ref, be_ref, o_ref):
    st = st_ref[...]
    mean = jnp.sum(st[:, 0, :], axis=0) * (1.0 / N)
    ex2 = jnp.sum(st[:, 1, :], axis=0) * (1.0 / N)
    var = ex2 - mean * mean
    inv = lax.rsqrt(var + 1e-5)
    o_ref[...] = jnp.maximum(
        (z2_ref[...] - mean[None]) * inv[None] * g_ref[...] + be_ref[...], 0.0)


def _bn_apply(z2, st, g, be):
    nb = 10
    return pl.pallas_call(
        _bn_k,
        grid=(nb,),
        in_specs=[pl.BlockSpec((N // nb, H), lambda i: (i, 0)),
                  pl.BlockSpec((nb, 2, H), lambda i: (0, 0, 0)),
                  pl.BlockSpec((1, H), lambda i: (0, 0)),
                  pl.BlockSpec((1, H), lambda i: (0, 0))],
        out_specs=pl.BlockSpec((N // nb, H), lambda i: (i, 0)),
        out_shape=jax.ShapeDtypeStruct((N, H), _f32),
    )(z2, st, g, be)


def _head_pre_k(h_ref, wa_ref, wb_ref, oa, ob):
    hh = h_ref[...]
    oa[...] = jnp.dot(hh, wa_ref[...], preferred_element_type=_f32)
    ob[...] = jnp.dot(hh, wb_ref[...], preferred_element_type=_f32)


def _head_pre(h, wa, wb):
    nb = 10
    return pl.pallas_call(
        _head_pre_k,
        grid=(nb,),
        in_specs=[pl.BlockSpec((N // nb, H), lambda i: (i, 0)),
                  pl.BlockSpec((H, H), lambda i: (0, 0)),
                  pl.BlockSpec((H, H), lambda i: (0, 0))],
        out_specs=[pl.BlockSpec((N // nb, H), lambda i: (i, 0))] * 2,
        out_shape=[jax.ShapeDtypeStruct((N, H), _f32)] * 2,
    )(h, wa, wb)


def _head_post_k(o1_ref, w2_ref, b2_ref, w3_ref, b3_ref, o_ref):
    t = jnp.maximum(
        jnp.dot(o1_ref[...], w2_ref[...], preferred_element_type=_f32)
        + b2_ref[...], 0.0)
    o_ref[...] = jnp.dot(t, w3_ref[...], preferred_element_type=_f32) + b3_ref[...]


def _head_post(o1, w2, b2, w3, b3):
    eb = 2000
    return pl.pallas_call(
        _head_post_k,
        grid=(E // eb,),
        in_specs=[pl.BlockSpec((eb, H), lambda i: (i, 0)),
                  pl.BlockSpec((H, H // 2), lambda i: (0, 0)),
                  pl.BlockSpec((1, H // 2), lambda i: (0, 0)),
                  pl.BlockSpec((H // 2, C), lambda i: (0, 0)),
                  pl.BlockSpec((1, C), lambda i: (0, 0))],
        out_specs=pl.BlockSpec((eb, C), lambda i: (i, 0)),
        out_shape=jax.ShapeDtypeStruct((E, C), _f32),
    )(o1, w2, b2, w3, b3)


_MESH = plsc.VectorSubcoreMesh(core_axis_name="c", subcore_axis_name="s",
                               num_cores=NC, num_subcores=NS)


def _num_chunks(w):
    rem = NCHUNK - (NCHUNK // NW) * NW
    return jnp.where(w < rem, NCHUNK // NW + 1, NCHUNK // NW).astype(jnp.int32)


_WSIZES = [10080] * 11 + [9840] * 4 + [9760]
_CUTS = [0]
for _ws in _WSIZES:
    _CUTS.append(_CUTS[-1] + _ws)
_CUTS = _CUTS + [160000 + _c for _c in _CUTS[1:]]
assert _CUTS[-1] == E and len(_CUTS) == NW + 1

NROWS_SH = N + 16

_ZTAIL = NROWS_SH - (NS - 1) * ROWS_PT


def _sc_aggr_body(h_hbm, e_hbm, perm_hbm, ss_hbm, ds_hbm, meta_hbm,
                  frow_hbm, z_hbm, out_hbm,
                  sidx, pidx, didx, ebuf, gbuf, accbuf, fragsrc, fragidx,
                  metav, aggr_sh, sem):
    c = lax.axis_index("c")
    s = lax.axis_index("s")
    w = c * NS + s

    pltpu.sync_copy(z_hbm.at[pl.ds(0, 16)], fragsrc)

    @pl.when(s < NS - 1)
    def _():
        for t in range(ROWS_PT // CB):
            pltpu.sync_copy(z_hbm, aggr_sh.at[pl.ds(s * ROWS_PT + t * CB, CB)])

    @pl.when(s == NS - 1)
    def _():
        for t in range(_ZTAIL // CB):
            pltpu.sync_copy(
                z_hbm, aggr_sh.at[pl.ds((NS - 1) * ROWS_PT + t * CB, CB)])
        zr = _ZTAIL - (_ZTAIL // CB) * CB
        if zr:
            pltpu.sync_copy(z_hbm.at[pl.ds(0, zr)],
                            aggr_sh.at[pl.ds(NROWS_SH - zr, zr)])

    pltpu.sync_copy(meta_hbm, metav)
    pltpu.sync_copy(frow_hbm.at[pl.ds(pl.multiple_of(w * 16, 16), 16)], fragidx)
    plsc.subcore_barrier()

    mv = metav[pl.ds(pl.multiple_of(w * 4, 4), 16)]
    lo = mv[0]
    hi = mv[1]
    first = mv[2]
    cont = mv[3] != 0
    nrows = hi - lo
    nchunks = lax.div(nrows + CB - 1, CB)

    zvec = jnp.zeros((16,), _f32)

    def _chunk(k, prev):
        base = pl.multiple_of(lo + k * CB, 16)
        rem = jnp.minimum(CB, hi - base)
        pltpu.sync_copy(ss_hbm.at[pl.ds(base, CB)], sidx)
        pltpu.sync_copy(ds_hbm.at[pl.ds(base, CB + 16)], didx)
        pltpu.sync_copy(perm_hbm.at[pl.ds(base, CB)], pidx)
        pltpu.async_copy(e_hbm.at[pidx], ebuf, sem).wait()
        pltpu.async_copy(h_hbm.at[sidx], gbuf, sem).wait()

        def _row(i, prev):
            cur = didx[pl.ds(i, 16)][0]
            same = cur == prev
            flush = jnp.logical_and(
                jnp.logical_not(same),
                jnp.logical_and(prev >= 0,
                                jnp.logical_not(
                                    jnp.logical_and(cont, prev == first))))
            save = jnp.logical_and(
                jnp.logical_not(same),
                jnp.logical_and(cont, prev == first))

            @pl.when(flush)
            def _():
                pltpu.sync_copy(accbuf, aggr_sh.at[pl.ds(prev, 1)])

            @pl.when(save)
            def _():
                for j in range(H // 16):
                    fragsrc[0, pl.ds(16 * j, 16)] = accbuf[0, pl.ds(16 * j, 16)]

            for j in range(H // 16):
                sl = pl.ds(16 * j, 16)
                m = jnp.maximum(ebuf[i, sl] + gbuf[i, sl], 0.0)
                accbuf[0, sl] = jnp.where(same, accbuf[0, sl], zvec) + m
            return cur

        return lax.fori_loop(0, rem, _row, prev)

    prev = lax.fori_loop(0, nchunks, _chunk, jnp.int32(-1))
    plsc.subcore_barrier()
    last_is_frag = jnp.logical_and(cont, prev == first)

    @pl.when(jnp.logical_and(prev >= 0, jnp.logical_not(last_is_frag)))
    def _():
        pltpu.sync_copy(accbuf, aggr_sh.at[pl.ds(prev, 1)])

    @pl.when(last_is_frag)
    def _():
        for j in range(H // 16):
            fragsrc[0, pl.ds(16 * j, 16)] = accbuf[0, pl.ds(16 * j, 16)]

    plsc.subcore_barrier()
    pltpu.sync_copy(fragsrc, aggr_sh.at[fragidx], add=True)
    plsc.subcore_barrier()

    @pl.when(s < NS - 1)
    def _():
        pltpu.sync_copy(aggr_sh.at[pl.ds(s * ROWS_PT, ROWS_PT)],
                        out_hbm.at[c, pl.ds(s * ROWS_PT, ROWS_PT)])

    @pl.when(s == NS - 1)
    def _():
        pltpu.sync_copy(aggr_sh.at[pl.ds((NS - 1) * ROWS_PT, N_TAIL)],
                        out_hbm.at[c, pl.ds((NS - 1) * ROWS_PT, N_TAIL)])


_sc_aggr = functools.partial(
    pl.kernel,
    out_type=jax.ShapeDtypeStruct((NC, N, H), _f32),
    mesh=_MESH,
    scratch_types=[
        pltpu.VMEM((CB,), jnp.int32),
        pltpu.VMEM((CB,), jnp.int32),
        pltpu.VMEM((CB + 16,), jnp.int32),
        pltpu.VMEM((CB, H), _f32),
        pltpu.VMEM((CB, H), _f32),
        pltpu.VMEM((1, H), _f32),
        pltpu.VMEM((16, H), _f32),
        pltpu.VMEM((16,), jnp.int32),
        pltpu.VMEM((NW * 4 + 16,), jnp.int32),
        pltpu.VMEM_SHARED((NROWS_SH, H), _f32),
        pltpu.SemaphoreType.DMA,
    ])(_sc_aggr_body)


def _sc_head_body(a_hbm, b_hbm, c_hbm, src_hbm, dst_hbm, out_hbm,
                  sidx, didx, cbuf, gbuf, g2buf, sem):
    c = lax.axis_index("c")
    s = lax.axis_index("s")
    w = s * NC + c

    def _chunk(k, carry):
        base = (w + k * NW) * CB
        pltpu.sync_copy(src_hbm.at[pl.ds(base, CB)], sidx)
        pltpu.sync_copy(dst_hbm.at[pl.ds(base, CB)], didx)
        pltpu.sync_copy(c_hbm.at[pl.ds(base, CB)], cbuf)
        pltpu.async_copy(a_hbm.at[sidx], gbuf, sem).wait()
        pltpu.async_copy(b_hbm.at[didx], g2buf, sem).wait()

        def _row(i, cc):
            for j in range(H // 16):
                sl = pl.ds(16 * j, 16)
                cbuf[i, sl] = jnp.maximum(
                    cbuf[i, sl] + gbuf[i, sl] + g2buf[i, sl], 0.0)
            return cc
        lax.fori_loop(0, CB, _row, 0)
        pltpu.sync_copy(cbuf, out_hbm.at[pl.ds(base, CB)])
        return carry

    lax.fori_loop(0, _num_chunks(w), _chunk, 0)


_sc_head = functools.partial(
    pl.kernel,
    out_type=jax.ShapeDtypeStruct((E, H), _f32),
    mesh=_MESH,
    scratch_types=[
        pltpu.VMEM((CB,), jnp.int32),
        pltpu.VMEM((CB,), jnp.int32),
        pltpu.VMEM((CB, H), _f32),
        pltpu.VMEM((CB, H), _f32),
        pltpu.VMEM((CB, H), _f32),
        pltpu.SemaphoreType.DMA,
    ])(_sc_head_body)


def kernel(x, edge_index, edge_attr, params):
    p = params
    src = edge_index[0]
    dst = edge_index[1]

    perm = jnp.argsort(dst, stable=True).astype(jnp.int32)
    ds_s = jnp.take(dst, perm)
    ss_s = jnp.take(src, perm)
    cuts = jnp.asarray(_CUTS, jnp.int32)
    firstnode = jnp.take(ds_s, cuts[:NW])
    contmask = (jnp.arange(NW, dtype=jnp.int32) % NS != 0) & (
        jnp.take(ds_s, jnp.maximum(cuts[:NW] - 1, 0)) == firstnode)
    meta = jnp.concatenate([
        jnp.stack([cuts[:NW], cuts[1:], firstnode,
                   contmask.astype(jnp.int32)], axis=1).reshape(NW * 4),
        jnp.zeros((16,), jnp.int32)])
    frow = jnp.full((NW, 16), N, jnp.int32).at[:, 0].set(
        jnp.where(contmask, firstnode, N)).reshape(NW * 16)
    zrows = jnp.zeros((CB, H), jnp.float32)
    pad_i = jnp.zeros((CB,), jnp.int32)
    perm_p = jnp.concatenate([perm, pad_i])
    ss_p = jnp.concatenate([ss_s, pad_i])
    ds_p = jnp.concatenate([ds_s, jnp.full((CB + 16,), N, jnp.int32)])

    wcat = jnp.concatenate(
        [lp['lin_w'] for lp in p['layers']] + [p['hw1'][2 * H:]], axis=1)
    bcat = jnp.concatenate(
        [lp['lin_b'] for lp in p['layers']] + [p['hb1']])[None]
    e0, e1, e2, ch = _edge_proj(edge_attr, wcat, bcat)
    es = (e0, e1, e2)

    h = _in_proj(x, p['in_w'], p['in_b'][None])
    for li, lp in enumerate(p['layers']):
        aggr = _sc_aggr(h, es[li], perm_p, ss_p, ds_p, meta, frow, zrows)
        z2, st = _node_mlp(h, aggr[0], aggr[1],
                           lp['w1'], lp['b1'][None], lp['w2'], lp['b2'][None])
        h = _bn_apply(z2, st, lp['gamma'][None], lp['beta'][None])

    a, b = _head_pre(h, p['hw1'][:H], p['hw1'][H:2 * H])
    o1 = _sc_head(a, b, ch, src, dst)
    return _head_post(o1, p['hw2'], p['hb2'][None], p['hw3'], p['hb3'][None])
